# baseline (speedup 1.0000x reference)
import functools

import jax
import jax.numpy as jnp
from jax import lax
from jax.experimental import pallas as pl
from jax.experimental.pallas import tpu as pltpu
from jax.experimental.pallas import tpu_sc as plsc

_NC = 2
_NS = 16


def kernel(tokens, spatial_shape, fc1_w, fc1_b, fc2_w, fc2_b):
    B, N, C = tokens.shape
    R = B * N
    flat = tokens.reshape(R, C)
    NW = _NC * _NS
    RPW = R // NW

    sv16 = jnp.pad(spatial_shape, (0, 14))

    mesh = plsc.VectorSubcoreMesh(core_axis_name="c", subcore_axis_name="s")

    @functools.partial(
        pl.kernel,
        out_type=jax.ShapeDtypeStruct((R, C), jnp.float32),
        mesh=mesh,
        scratch_types=[
            pltpu.VMEM((16,), jnp.int32),
            pltpu.VMEM((RPW, C), jnp.float32),
            pltpu.SemaphoreType.DMA((4,)),
            pltpu.SemaphoreType.DMA((4,)),
        ],
    )
    def sc_copy(sv_hbm, tok_hbm, out_hbm, sv_v, buf_v, sem_in, sem_out):
        wid = lax.axis_index("s") * _NC + lax.axis_index("c")
        base = wid * RPW
        CH = RPW // 4
        pltpu.sync_copy(sv_hbm, sv_v)
        sv = sv_v[...]
        valid = sv[0] * sv[1] == N

        @pl.when(valid)
        def _copy():
            ins = [
                pltpu.async_copy(
                    tok_hbm.at[pl.ds(base + k * CH, CH)],
                    buf_v.at[pl.ds(k * CH, CH)],
                    sem_in.at[k],
                )
                for k in range(4)
            ]
            outs = []
            for k in range(4):
                ins[k].wait()
                outs.append(
                    pltpu.async_copy(
                        buf_v.at[pl.ds(k * CH, CH)],
                        out_hbm.at[pl.ds(base + k * CH, CH)],
                        sem_out.at[k],
                    )
                )
            for k in range(4):
                outs[k].wait()

        @pl.when(jnp.logical_not(valid))
        def _nan_fill():
            nanv = jnp.full((16,), jnp.nan, jnp.float32)

            def fill_row(i, _):
                def fill_seg(j, _):
                    buf_v[i, pl.ds(j * 16, 16)] = nanv
                    return 0
                return lax.fori_loop(0, C // 16, fill_seg, 0)

            lax.fori_loop(0, RPW, fill_row, 0)
            pltpu.sync_copy(buf_v, out_hbm.at[pl.ds(base, RPW)])

    out = sc_copy(sv16, flat)
    return out.reshape(B, N, C)


# speedup vs baseline: 3.5320x; 3.5320x over previous
import jax
import jax.numpy as jnp
from jax.experimental import pallas as pl
from jax.experimental.pallas import tpu as pltpu


def kernel(tokens, spatial_shape, fc1_w, fc1_b, fc2_w, fc2_b):
    B, N, C = tokens.shape
    flat = tokens.reshape(B * N, C)
    R = B * N
    BLK = 2048

    def body(sv_ref, x_ref, o_ref):
        valid = sv_ref[0] * sv_ref[1] == N
        o_ref[...] = jnp.where(valid, x_ref[...], jnp.float32(jnp.nan))

    out = pl.pallas_call(
        body,
        grid=(R // BLK,),
        in_specs=[
            pl.BlockSpec(memory_space=pltpu.MemorySpace.SMEM),
            pl.BlockSpec((BLK, C), lambda i: (i, 0)),
        ],
        out_specs=pl.BlockSpec((BLK, C), lambda i: (i, 0)),
        out_shape=jax.ShapeDtypeStruct((R, C), jnp.float32),
    )(spatial_shape, flat)
    return out.reshape(B, N, C)


# speedup vs baseline: 3.5323x; 1.0001x over previous
import jax
import jax.numpy as jnp
from jax.experimental import pallas as pl
from jax.experimental.pallas import tpu as pltpu


def kernel(tokens, spatial_shape, fc1_w, fc1_b, fc2_w, fc2_b):
    B, N, C = tokens.shape
    flat = tokens.reshape(B * N, C)
    R = B * N
    BLK = 2048

    def body(sv_ref, x_ref, o_ref):
        valid = sv_ref[0] * sv_ref[1] == N
        o_ref[...] = jnp.where(valid, x_ref[...], jnp.float32(jnp.nan))

    out = pl.pallas_call(
        body,
        grid=(R // BLK,),
        in_specs=[
            pl.BlockSpec(memory_space=pltpu.MemorySpace.SMEM),
            pl.BlockSpec((BLK, C), lambda i: (i, 0)),
        ],
        out_specs=pl.BlockSpec((BLK, C), lambda i: (i, 0)),
        out_shape=jax.ShapeDtypeStruct((R, C), jnp.float32),
        compiler_params=pltpu.CompilerParams(dimension_semantics=("arbitrary",)),
    )(spatial_shape, flat)
    return out.reshape(B, N, C)
